# Initial kernel scaffold; baseline (speedup 1.0000x reference)
import jax
import jax.numpy as jnp
from jax.experimental import pallas as pl


def kernel(x, gate_W, sW1, sb1, sW2, sb2, rW1, rb1, rW2, rb2):
    raise NotImplementedError("write your pallas kernel here")


# speedup vs baseline: 1.6650x; 1.6650x over previous
import jax
import jax.numpy as jnp
from jax.experimental import pallas as pl
from jax.experimental.pallas import tpu as pltpu

HID = 768
INTER = 1024
NUM_ROUTED = 24
NUM_SHARED = 2
TOP_K = 6
RINTER = 256
N_SHARED_CHUNK = NUM_SHARED * (INTER // RINTER)
N_CHUNK = NUM_ROUTED + N_SHARED_CHUNK


def _router_kernel(xf_ref, gwt_ref, w_ref):
    logits = jnp.dot(xf_ref[...], gwt_ref[...], preferred_element_type=jnp.float32)
    scores = jax.nn.sigmoid(logits)
    n, e = scores.shape
    col = jax.lax.broadcasted_iota(jnp.int32, (n, e), 1)
    s = scores
    mask = jnp.zeros(scores.shape, dtype=jnp.bool_)
    for _ in range(TOP_K):
        m = jnp.max(s, axis=1, keepdims=True)
        is_max = s == m
        min_idx = jnp.min(jnp.where(is_max, col, e), axis=1, keepdims=True)
        pick = col == min_idx
        mask = mask | pick
        s = jnp.where(pick, -jnp.inf, s)
    sel = jnp.where(mask, scores, 0.0)
    w_ref[...] = sel / (jnp.sum(sel, axis=1, keepdims=True) + 1e-9)


def _moe_kernel(w_ref, x_ref, w1_ref, b1_ref, w2_ref, b2_ref, out_ref):
    c = pl.program_id(0)

    @pl.when(c == 0)
    def _():
        out_ref[...] = jnp.zeros_like(out_ref)

    h = jnp.dot(x_ref[...], w1_ref[0], preferred_element_type=jnp.float32) + b1_ref[0]
    h = jax.nn.gelu(h)
    y = jnp.dot(h, w2_ref[0], preferred_element_type=jnp.float32) + b2_ref[0]
    out_ref[...] += w_ref[0] * y


def kernel(x, gate_W, sW1, sb1, sW2, sb2, rW1, rb1, rW2, rb2):
    b, s, d = x.shape
    xf = x.reshape(-1, d)
    n = xf.shape[0]

    w_routed = pl.pallas_call(
        _router_kernel,
        out_shape=jax.ShapeDtypeStruct((n, NUM_ROUTED), jnp.float32),
    )(xf, gate_W.T)

    sW1c = sW1.reshape(NUM_SHARED, HID, INTER // RINTER, RINTER)
    sW1c = sW1c.transpose(0, 2, 1, 3).reshape(N_SHARED_CHUNK, HID, RINTER)
    sb1c = sb1.reshape(N_SHARED_CHUNK, RINTER)
    sW2c = sW2.reshape(N_SHARED_CHUNK, RINTER, HID)
    sb2c = jnp.repeat(sb2 / (INTER // RINTER), INTER // RINTER, axis=0)

    cW1 = jnp.concatenate([rW1, sW1c], axis=0)
    cb1 = jnp.concatenate([rb1, sb1c], axis=0).reshape(N_CHUNK, 1, RINTER)
    cW2 = jnp.concatenate([rW2, sW2c], axis=0)
    cb2 = jnp.concatenate([rb2, sb2c], axis=0).reshape(N_CHUNK, 1, HID)
    w_full = jnp.concatenate(
        [w_routed, jnp.ones((n, N_SHARED_CHUNK), jnp.float32)], axis=1
    )
    w_full = w_full.T.reshape(N_CHUNK, n, 1)

    out = pl.pallas_call(
        _moe_kernel,
        grid=(N_CHUNK,),
        in_specs=[
            pl.BlockSpec((1, n, 1), lambda c: (c, 0, 0)),
            pl.BlockSpec((n, HID), lambda c: (0, 0)),
            pl.BlockSpec((1, HID, RINTER), lambda c: (c, 0, 0)),
            pl.BlockSpec((1, 1, RINTER), lambda c: (c, 0, 0)),
            pl.BlockSpec((1, RINTER, HID), lambda c: (c, 0, 0)),
            pl.BlockSpec((1, 1, HID), lambda c: (c, 0, 0)),
        ],
        out_specs=pl.BlockSpec((n, HID), lambda c: (0, 0)),
        out_shape=jax.ShapeDtypeStruct((n, HID), jnp.float32),
    )(w_full, xf, cW1, cb1, cW2, cb2)

    aux_loss = jnp.asarray(0.0, dtype=jnp.float32)
    return (out.reshape(b, s, d), aux_loss)
